# baseline (speedup 1.0000x reference)
import functools

import jax
import jax.numpy as jnp
from jax import lax
from jax.experimental import pallas as pl
from jax.experimental.pallas import tpu as pltpu
from jax.experimental.pallas import tpu_sc as plsc

_B = 16384
_D = 32
_L = 16
_NC, _NS = 2, 16
_NW = _NC * _NS
_RPW = _B // _NW
_CH = 128
_NCH = _RPW // _CH
_MAX_NORM = 2.0


def _rsqrt(x):
    i = plsc.bitcast(x, jnp.int32)
    i = jnp.int32(0x5F3759DF) - (i >> 1)
    y = plsc.bitcast(i, jnp.float32)
    for _ in range(3):
        y = y * (jnp.float32(1.5) - jnp.float32(0.5) * x * y * y)
    return y


def _scale(n2):
    y = _rsqrt(jnp.maximum(n2, jnp.float32(1e-12)))
    return jnp.minimum(jnp.float32(1.0), jnp.float32(_MAX_NORM) * y)


_mesh = plsc.VectorSubcoreMesh(core_axis_name="c", subcore_axis_name="s")


@functools.partial(
    pl.kernel,
    mesh=_mesh,
    out_type=jax.ShapeDtypeStruct((_B,), jnp.float32),
    compiler_params=pltpu.CompilerParams(
        needs_layout_passes=False, use_tc_tiling_on_sc=False),
    scratch_types=[
        pltpu.VMEM((_NCH, _CH), jnp.int32),
        pltpu.VMEM((_NCH, _CH), jnp.int32),
        pltpu.VMEM((_RPW, _D), jnp.float32),
        pltpu.VMEM((_RPW, _D), jnp.float32),
        pltpu.VMEM((_RPW,), jnp.float32),
        pltpu.SemaphoreType.DMA,
    ],
)
def _als_logits(u_hbm, v_hbm, users_hbm, items_hbm, out_hbm,
                uidx, vidx, ue, ve, outv, sem):
    wid = lax.axis_index("s") * _NC + lax.axis_index("c")
    pltpu.sync_copy(u_hbm.at[pl.ds(wid * _NCH, _NCH)], uidx)
    pltpu.sync_copy(v_hbm.at[pl.ds(wid * _NCH, _NCH)], vidx)

    copies = []
    for c in range(_NCH):
        copies.append(pltpu.async_copy(
            users_hbm.at[uidx.at[c]], ue.at[pl.ds(c * _CH, _CH)], sem))
        copies.append(pltpu.async_copy(
            items_hbm.at[vidx.at[c]], ve.at[pl.ds(c * _CH, _CH)], sem))
    for cp in copies:
        cp.wait()

    def group(g, carry):
        rid = g * _L + lax.iota(jnp.int32, _L)
        uv = jnp.zeros((_L,), jnp.float32)
        uu = jnp.zeros((_L,), jnp.float32)
        vv = jnp.zeros((_L,), jnp.float32)
        for d in range(_D):
            dcol = jnp.full((_L,), d, jnp.int32)
            a = plsc.load_gather(ue, [rid, dcol])
            b = plsc.load_gather(ve, [rid, dcol])
            uv = uv + a * b
            uu = uu + a * a
            vv = vv + b * b
        outv[pl.ds(g * _L, _L)] = uv * _scale(uu) * _scale(vv)
        return carry

    lax.fori_loop(0, _RPW // _L, group, 0)
    pltpu.sync_copy(outv, out_hbm.at[pl.ds(wid * _RPW, _RPW)])


def kernel(u, v, users_table, items_table):
    u2 = u.astype(jnp.int32).reshape(_NW * _NCH, _CH)
    v2 = v.astype(jnp.int32).reshape(_NW * _NCH, _CH)
    return _als_logits(u2, v2, users_table, items_table)


# speedup vs baseline: 1.3591x; 1.3591x over previous
import functools

import jax
import jax.numpy as jnp
from jax import lax
from jax.experimental import pallas as pl
from jax.experimental.pallas import tpu as pltpu
from jax.experimental.pallas import tpu_sc as plsc

_B = 16384
_D = 32
_L = 16
_NC, _NS = 2, 16
_NW = _NC * _NS
_RPW = _B // _NW
_NCHUNK = _RPW // _L
_MAX_NORM = 2.0


def _rsqrt(x):
    i = plsc.bitcast(x, jnp.int32)
    i = jnp.int32(0x5F3759DF) - (i >> 1)
    y = plsc.bitcast(i, jnp.float32)
    for _ in range(3):
        y = y * (jnp.float32(1.5) - jnp.float32(0.5) * x * y * y)
    return y


def _scale(n2):
    y = _rsqrt(jnp.maximum(n2, jnp.float32(1e-12)))
    return jnp.minimum(jnp.float32(1.0), jnp.float32(_MAX_NORM) * y)


_mesh = plsc.VectorSubcoreMesh(core_axis_name="c", subcore_axis_name="s")


@functools.partial(
    pl.kernel,
    mesh=_mesh,
    out_type=jax.ShapeDtypeStruct((_B,), jnp.float32),
    compiler_params=pltpu.CompilerParams(
        needs_layout_passes=False, use_tc_tiling_on_sc=True),
    scratch_types=[
        pltpu.VMEM((_RPW,), jnp.int32),
        pltpu.VMEM((_RPW,), jnp.int32),
        pltpu.VMEM((_L, 8, _D), jnp.float32),
        pltpu.VMEM((_L, 8, _D), jnp.float32),
        pltpu.VMEM((_RPW * _D,), jnp.float32),
        pltpu.VMEM((_RPW * _D,), jnp.float32),
        pltpu.VMEM((_RPW,), jnp.float32),
        pltpu.SemaphoreType.DMA,
        pltpu.SemaphoreType.DMA,
    ],
)
def _als_logits(u_hbm, v_hbm, users_hbm, items_hbm, out_hbm,
                uidx, vidx, uring, vring, ue, ve, outv, usem, vsem):
    wid = lax.axis_index("s") * _NC + lax.axis_index("c")
    base = wid * _RPW
    pltpu.sync_copy(u_hbm.at[pl.ds(base, _RPW)], uidx)
    pltpu.sync_copy(v_hbm.at[pl.ds(base, _RPW)], vidx)

    def fetch_chunk(ch, carry):
        uvec = uidx[pl.ds(ch * _L, _L)]
        vvec = vidx[pl.ds(ch * _L, _L)]
        ucps, vcps = [], []
        for j in range(_L):
            gu = pl.multiple_of((uvec[j] >> 3) * 8, 8)
            gv = pl.multiple_of((vvec[j] >> 3) * 8, 8)
            ucps.append(pltpu.async_copy(
                users_hbm.at[pl.ds(gu, 8), :], uring.at[j], usem))
            vcps.append(pltpu.async_copy(
                items_hbm.at[pl.ds(gv, 8), :], vring.at[j], vsem))
        for j in range(_L):
            i = ch * _L + j
            ucps[j].wait()
            ru = uvec[j] & 7
            ue[pl.ds(i * _D, _L)] = uring[j, ru, pl.ds(0, _L)]
            ue[pl.ds(i * _D + _L, _L)] = uring[j, ru, pl.ds(_L, _L)]
            vcps[j].wait()
            rv = vvec[j] & 7
            ve[pl.ds(i * _D, _L)] = vring[j, rv, pl.ds(0, _L)]
            ve[pl.ds(i * _D + _L, _L)] = vring[j, rv, pl.ds(_L, _L)]
        return carry

    lax.fori_loop(0, _NCHUNK, fetch_chunk, 0)

    def group(g, carry):
        flat = (g * _L + lax.iota(jnp.int32, _L)) * _D
        uv = jnp.zeros((_L,), jnp.float32)
        uu = jnp.zeros((_L,), jnp.float32)
        vv = jnp.zeros((_L,), jnp.float32)
        for d in range(_D):
            a = plsc.load_gather(ue, [flat + d])
            b = plsc.load_gather(ve, [flat + d])
            uv = uv + a * b
            uu = uu + a * a
            vv = vv + b * b
        outv[pl.ds(g * _L, _L)] = uv * _scale(uu) * _scale(vv)
        return carry

    lax.fori_loop(0, _NCHUNK, group, 0)
    pltpu.sync_copy(outv, out_hbm.at[pl.ds(base, _RPW)])


def kernel(u, v, users_table, items_table):
    return _als_logits(u.astype(jnp.int32), v.astype(jnp.int32),
                       users_table, items_table)
